# baseline (speedup 1.0000x reference)
import functools

import jax
import jax.numpy as jnp
from jax import lax
from jax.experimental import pallas as pl
from jax.experimental.pallas import tpu as pltpu
from jax.experimental.pallas import tpu_sc as plsc

VOCAB_N = 100000
HIDDEN_N = 768
MAX_POS_N = 4096
BATCH_N = 4
SEQ_N = 4096

_NC = 2
_NS = 16
_NW = _NC * _NS
_L = 16

_B = BATCH_N * SEQ_N
_BPW = _B // _NW
_C = 64
_NCHUNK = _BPW // _C
_NV = HIDDEN_N // _L


def _scaledsin_table():
    pos = jnp.arange(MAX_POS_N, dtype=jnp.float32)
    half_d = HIDDEN_N // 2
    freq_seq = -jnp.arange(half_d, dtype=jnp.float32) / float(half_d)
    inv_freq = 10000.0 ** freq_seq
    sinusoid = pos[:, None] * inv_freq[None, :]
    return jnp.concatenate([jnp.sin(sinusoid), jnp.cos(sinusoid)], axis=-1)


def _sc_embed(ids_flat, table, posemb, scale16):
    mesh = plsc.VectorSubcoreMesh(core_axis_name="c", subcore_axis_name="s")

    @functools.partial(
        pl.kernel,
        out_type=jax.ShapeDtypeStruct((_B, HIDDEN_N), jnp.float32),
        mesh=mesh,
        scratch_types=[
            pltpu.VMEM((_C,), jnp.int32),
            pltpu.VMEM((_C, HIDDEN_N), jnp.float32),
            pltpu.VMEM((_C, HIDDEN_N), jnp.float32),
            pltpu.VMEM((_L,), jnp.float32),
            pltpu.SemaphoreType.DMA,
        ],
    )
    def k(ids_hbm, tab_hbm, pos_hbm, scale_hbm, out_hbm,
          idx_v, rows_v, pos_v, scale_v, sem):
        wid = lax.axis_index("s") * _NC + lax.axis_index("c")
        base = wid * _BPW
        pltpu.sync_copy(scale_hbm, scale_v)
        sv = scale_v[...]

        def chunk_body(c, carry):
            cbase = base + c * _C
            sbase = lax.rem(cbase, SEQ_N)
            pltpu.sync_copy(ids_hbm.at[pl.ds(cbase, _C)], idx_v)
            gather = pltpu.async_copy(tab_hbm.at[idx_v], rows_v, sem)
            pltpu.sync_copy(pos_hbm.at[pl.ds(sbase, _C)], pos_v)
            gather.wait()

            def row_body(r, c2):
                for j in range(_NV):
                    sl = pl.ds(j * _L, _L)
                    plsc.addupdate(rows_v.at[r, sl], pos_v[r, sl] * sv)
                return c2

            lax.fori_loop(0, _C, row_body, 0)
            pltpu.sync_copy(rows_v, out_hbm.at[pl.ds(cbase, _C)])
            return carry

        lax.fori_loop(0, _NCHUNK, chunk_body, 0)

    return k(ids_flat, table, posemb, scale16)


def kernel(input_ids, word_embeddings, scale):
    ids_flat = input_ids.reshape(-1).astype(jnp.int32)
    posemb = _scaledsin_table()
    scale16 = jnp.broadcast_to(scale.astype(jnp.float32), (_L,))
    out = _sc_embed(ids_flat, word_embeddings, posemb, scale16)
    return out.reshape(BATCH_N, SEQ_N, HIDDEN_N)


# speedup vs baseline: 1.1245x; 1.1245x over previous
import functools

import jax
import jax.numpy as jnp
from jax import lax
from jax.experimental import pallas as pl
from jax.experimental.pallas import tpu as pltpu
from jax.experimental.pallas import tpu_sc as plsc

VOCAB_N = 100000
HIDDEN_N = 768
MAX_POS_N = 4096
BATCH_N = 4
SEQ_N = 4096

_NC = 2
_NS = 16
_NW = _NC * _NS
_L = 16

_B = BATCH_N * SEQ_N
_PPW = SEQ_N // _NW
_C = 64
_NJ = _PPW // _C
_NV = HIDDEN_N // _L


def _scaledsin_table():
    pos = jnp.arange(MAX_POS_N, dtype=jnp.float32)
    half_d = HIDDEN_N // 2
    freq_seq = -jnp.arange(half_d, dtype=jnp.float32) / float(half_d)
    inv_freq = 10000.0 ** freq_seq
    sinusoid = pos[:, None] * inv_freq[None, :]
    return jnp.concatenate([jnp.sin(sinusoid), jnp.cos(sinusoid)], axis=-1)


def _sc_embed(ids_flat, table, posemb, scale16):
    mesh = plsc.VectorSubcoreMesh(core_axis_name="c", subcore_axis_name="s")

    @functools.partial(
        pl.kernel,
        out_type=jax.ShapeDtypeStruct((_B, HIDDEN_N), jnp.float32),
        mesh=mesh,
        scratch_types=[
            pltpu.VMEM((_C,), jnp.int32),
            pltpu.VMEM((_C, HIDDEN_N), jnp.float32),
            pltpu.VMEM((_C, HIDDEN_N), jnp.float32),
            pltpu.VMEM((_L,), jnp.float32),
            pltpu.SemaphoreType.DMA,
        ],
    )
    def k(ids_hbm, tab_hbm, pos_hbm, scale_hbm, out_hbm,
          idx_v, rows_v, pos_v, scale_v, sem):
        wid = lax.axis_index("s") * _NC + lax.axis_index("c")
        pbase = wid * _PPW
        pltpu.sync_copy(scale_hbm, scale_v)
        sv = scale_v[...]

        def chunk_body(k, carry):
            j = k // BATCH_N
            b = lax.rem(k, BATCH_N)
            sbase = pbase + j * _C
            cbase = b * SEQ_N + sbase

            @pl.when(b == 0)
            def _():
                pltpu.sync_copy(pos_hbm.at[pl.ds(sbase, _C)], pos_v)

            pltpu.sync_copy(ids_hbm.at[pl.ds(cbase, _C)], idx_v)
            pltpu.async_copy(tab_hbm.at[idx_v], rows_v, sem).wait()

            def row_body(r, c2):
                for v in range(_NV):
                    sl = pl.ds(v * _L, _L)
                    plsc.addupdate(rows_v.at[r, sl], pos_v[r, sl] * sv)
                return c2

            lax.fori_loop(0, _C, row_body, 0)
            pltpu.sync_copy(rows_v, out_hbm.at[pl.ds(cbase, _C)])
            return carry

        lax.fori_loop(0, _NJ * BATCH_N, chunk_body, 0)

    return k(ids_flat, table, posemb, scale16)


def kernel(input_ids, word_embeddings, scale):
    ids_flat = input_ids.reshape(-1).astype(jnp.int32)
    posemb = _scaledsin_table()
    scale16 = jnp.broadcast_to(scale.astype(jnp.float32), (_L,))
    out = _sc_embed(ids_flat, word_embeddings, posemb, scale16)
    return out.reshape(BATCH_N, SEQ_N, HIDDEN_N)
